# baseline (speedup 1.0000x reference)
import functools

import jax
import jax.numpy as jnp
import numpy as np
from jax import lax
from jax.experimental import pallas as pl
from jax.experimental.pallas import tpu as pltpu
from jax.experimental.pallas import tpu_sc as plsc

H_DIM = 32
W_DIM = 32
HW = H_DIM * W_DIM

NC = 2
NS = 16
NW = NC * NS
L = 16

CPU = 32
HC = 4
S_CHUNK = HC * W_DIM
N_CHUNK = H_DIM // HC


def _diag_perm(mode):
    idx = []
    for d in range(H_DIM + W_DIM - 1):
        for i in range(H_DIM):
            j = d - i if mode == "rd" else i - (H_DIM - 1 - d)
            if 0 <= j < W_DIM:
                idx.append(i * W_DIM + j)
    return np.asarray(idx, dtype=np.int32)


def _inv(p):
    inv = np.empty_like(p)
    inv[p] = np.arange(p.size, dtype=p.dtype)
    return inv


_RD_INV = _inv(_diag_perm("rd"))
_LD_INV = _inv(_diag_perm("ld"))


@functools.partial(jax.jit, static_argnums=(3, 4))
def _diag_scan(x6, rd_inv, ld_inv, b_dim, c_dim):
    rows = b_dim * c_dim
    upb = c_dim // CPU
    steps = upb * N_CHUNK
    mesh = plsc.VectorSubcoreMesh(core_axis_name="c", subcore_axis_name="s")

    @functools.partial(
        pl.kernel,
        mesh=mesh,
        compiler_params=pltpu.CompilerParams(
            needs_layout_passes=False, use_tc_tiling_on_sc=True
        ),
        out_type=(
            jax.ShapeDtypeStruct((rows, HW), jnp.float32),
            jax.ShapeDtypeStruct((rows, HW), jnp.float32),
        ),
        scratch_types=[
            pltpu.VMEM((HW,), jnp.int32),
            pltpu.VMEM((HW,), jnp.int32),
            pltpu.VMEM((HC, 4, 8, 128), jnp.float32),
            pltpu.VMEM((HC, 4, 8, 128), jnp.float32),
            pltpu.VMEM((CPU, HW), jnp.float32),
            pltpu.VMEM((CPU, HW), jnp.float32),
            pltpu.SemaphoreType.DMA,
            pltpu.SemaphoreType.DMA,
            pltpu.SemaphoreType.DMA,
            pltpu.SemaphoreType.DMA,
        ],
    )
    def k(x_hbm, rdi_hbm, ldi_hbm, yrd_hbm, yld_hbm,
          rd_v, ld_v, in0, in1, or_v, ol_v,
          in_s0, in_s1, or_s, ol_s):
        b = lax.axis_index("s") * NC + lax.axis_index("c")
        pltpu.sync_copy(rdi_hbm, rd_v)
        pltpu.sync_copy(ldi_hbm, ld_v)

        def start_in(step, in_v, in_s):
            sc = jnp.minimum(step, steps - 1)
            cq = sc // N_CHUNK
            hc = sc % N_CHUNK
            pltpu.async_copy(
                x_hbm.at[b, pl.ds(hc * HC, HC), slice(None), cq // 4], in_v, in_s
            )

        iota = lax.iota(jnp.int32, L)

        def scatter_chunk(step, in_v):
            cq = step // N_CHUNK
            hc = step % N_CHUNK
            lane0 = (cq % 4) * CPU
            s0 = hc * S_CHUNK

            @plsc.parallel_loop(0, S_CHUNK)
            def _(sl):
                hloc = sl // W_DIM
                w8 = (sl % W_DIM) // 8
                wq = sl % 8
                sv = jnp.full((L,), s0 + sl, jnp.int32)
                kr = plsc.load_gather(rd_v, [sv])
                kl = plsc.load_gather(ld_v, [sv])
                for g in range(CPU // L):
                    rowi = iota + jnp.int32(g * L)
                    v = in_v[hloc, w8, wq, pl.ds(lane0 + g * L, L)]
                    plsc.store_scatter(or_v, [rowi, kr], v)
                    plsc.store_scatter(ol_v, [rowi, kl], v)

        def flush(cq):
            row0 = b * c_dim + cq * CPU
            pltpu.async_copy(or_v, yrd_hbm.at[pl.ds(row0, CPU)], or_s)
            pltpu.async_copy(ol_v, yld_hbm.at[pl.ds(row0, CPU)], ol_s)

        def wait_flush():
            pltpu.make_async_copy(or_v, yrd_hbm.at[pl.ds(0, CPU)], or_s).wait()
            pltpu.make_async_copy(ol_v, yld_hbm.at[pl.ds(0, CPU)], ol_s).wait()

        start_in(0, in0, in_s0)
        start_in(1, in1, in_s1)

        def half(step, in_v, in_s):
            pltpu.make_async_copy(
                x_hbm.at[b, pl.ds(0, HC), slice(None), 0], in_v, in_s
            ).wait()

            @pl.when(jnp.logical_and(step % N_CHUNK == 0, step > 0))
            def _():
                wait_flush()

            scatter_chunk(step, in_v)
            start_in(step + 2, in_v, in_s)

            @pl.when(step % N_CHUNK == N_CHUNK - 1)
            def _():
                flush(step // N_CHUNK)

        def body(i, carry):
            half(2 * i, in0, in_s0)
            half(2 * i + 1, in1, in_s1)
            return carry

        lax.fori_loop(0, steps // 2, body, 0)

        pltpu.make_async_copy(x_hbm.at[b, pl.ds(0, HC), slice(None), 0], in0, in_s0).wait()
        pltpu.make_async_copy(x_hbm.at[b, pl.ds(0, HC), slice(None), 0], in1, in_s1).wait()
        wait_flush()

    return k(x6, rd_inv, ld_inv)


def kernel(x):
    B, C, H, W = x.shape
    x6 = (
        jnp.transpose(x, (0, 2, 3, 1))
        .reshape(B, H, 4, 8, C // 128, 128)
        .transpose(0, 1, 2, 4, 3, 5)
    )
    yrd, yld = _diag_scan(x6, jnp.asarray(_RD_INV), jnp.asarray(_LD_INV), B, C)
    return yrd.reshape(B, C, HW), yld.reshape(B, C, HW)


# speedup vs baseline: 2.7503x; 2.7503x over previous
import functools

import jax
import jax.numpy as jnp
import numpy as np
from jax import lax
from jax.experimental import pallas as pl
from jax.experimental.pallas import tpu as pltpu
from jax.experimental.pallas import tpu_sc as plsc

H_DIM = 32
W_DIM = 32
HW = H_DIM * W_DIM

NC = 2
NS = 16
NW = NC * NS
L = 16

CPU = 32
HC = 4
S_CHUNK = HC * W_DIM
N_CHUNK = H_DIM // HC


def _diag_perm(mode):
    idx = []
    for d in range(H_DIM + W_DIM - 1):
        for i in range(H_DIM):
            j = d - i if mode == "rd" else i - (H_DIM - 1 - d)
            if 0 <= j < W_DIM:
                idx.append(i * W_DIM + j)
    return np.asarray(idx, dtype=np.int32)


def _inv(p):
    inv = np.empty_like(p)
    inv[p] = np.arange(p.size, dtype=p.dtype)
    return inv


_RD_INV = _inv(_diag_perm("rd"))
_LD_INV = _inv(_diag_perm("ld"))


@functools.partial(jax.jit, static_argnums=(3, 4))
def _diag_scan(x6, rd_inv, ld_inv, b_dim, c_dim):
    rows = b_dim * c_dim
    upb = c_dim // CPU
    steps = upb * N_CHUNK
    mesh = plsc.VectorSubcoreMesh(core_axis_name="c", subcore_axis_name="s")

    @functools.partial(
        pl.kernel,
        mesh=mesh,
        compiler_params=pltpu.CompilerParams(
            needs_layout_passes=False, use_tc_tiling_on_sc=True
        ),
        out_type=(
            jax.ShapeDtypeStruct((rows, HW), jnp.float32),
            jax.ShapeDtypeStruct((rows, HW), jnp.float32),
        ),
        scratch_types=[
            pltpu.VMEM((HW,), jnp.int32),
            pltpu.VMEM((HW,), jnp.int32),
            pltpu.VMEM((HC, 4, 8, 128), jnp.float32),
            pltpu.VMEM((HC, 4, 8, 128), jnp.float32),
            pltpu.VMEM((CPU, HW), jnp.float32),
            pltpu.VMEM((CPU, HW), jnp.float32),
            pltpu.SemaphoreType.DMA,
            pltpu.SemaphoreType.DMA,
            pltpu.SemaphoreType.DMA,
            pltpu.SemaphoreType.DMA,
        ],
    )
    def k(x_hbm, rdi_hbm, ldi_hbm, yrd_hbm, yld_hbm,
          rd_v, ld_v, in0, in1, or_v, ol_v,
          in_s0, in_s1, or_s, ol_s):
        b = lax.axis_index("s") * NC + lax.axis_index("c")
        pltpu.sync_copy(rdi_hbm, rd_v)
        pltpu.sync_copy(ldi_hbm, ld_v)

        def start_in(step, in_v, in_s):
            sc = jnp.minimum(step, steps - 1)
            cq = sc // N_CHUNK
            hc = sc % N_CHUNK
            pltpu.async_copy(
                x_hbm.at[b, pl.ds(hc * HC, HC), slice(None), cq // 4], in_v, in_s
            )

        iota = lax.iota(jnp.int32, L)

        def scatter_chunk(step, in_v):
            cq = step // N_CHUNK
            hc = step % N_CHUNK
            lane0 = (cq % 4) * CPU
            s0 = hc * S_CHUNK

            @plsc.parallel_loop(0, S_CHUNK // L)
            def _(r):
                slv = jnp.int32(r * L) + iota
                hv = lax.shift_right_logical(slv, 5)
                w8v = jnp.bitwise_and(lax.shift_right_logical(slv, 3), 3)
                wqv = jnp.bitwise_and(slv, 7)
                kr = rd_v[pl.ds(s0 + r * L, L)]
                kl = ld_v[pl.ds(s0 + r * L, L)]
                for c0 in range(CPU):
                    cv = jnp.bitwise_and(jnp.int32(c0) + iota, CPU - 1)
                    v = plsc.load_gather(in_v, [hv, w8v, wqv, cv + lane0])
                    plsc.store_scatter(or_v, [cv, kr], v)
                    plsc.store_scatter(ol_v, [cv, kl], v)

        def flush(cq):
            row0 = b * c_dim + cq * CPU
            pltpu.async_copy(or_v, yrd_hbm.at[pl.ds(row0, CPU)], or_s)
            pltpu.async_copy(ol_v, yld_hbm.at[pl.ds(row0, CPU)], ol_s)

        def wait_flush():
            pltpu.make_async_copy(or_v, yrd_hbm.at[pl.ds(0, CPU)], or_s).wait()
            pltpu.make_async_copy(ol_v, yld_hbm.at[pl.ds(0, CPU)], ol_s).wait()

        start_in(0, in0, in_s0)
        start_in(1, in1, in_s1)

        def half(step, in_v, in_s):
            pltpu.make_async_copy(
                x_hbm.at[b, pl.ds(0, HC), slice(None), 0], in_v, in_s
            ).wait()

            @pl.when(jnp.logical_and(step % N_CHUNK == 0, step > 0))
            def _():
                wait_flush()

            scatter_chunk(step, in_v)
            start_in(step + 2, in_v, in_s)

            @pl.when(step % N_CHUNK == N_CHUNK - 1)
            def _():
                flush(step // N_CHUNK)

        def body(i, carry):
            half(2 * i, in0, in_s0)
            half(2 * i + 1, in1, in_s1)
            return carry

        lax.fori_loop(0, steps // 2, body, 0)

        pltpu.make_async_copy(x_hbm.at[b, pl.ds(0, HC), slice(None), 0], in0, in_s0).wait()
        pltpu.make_async_copy(x_hbm.at[b, pl.ds(0, HC), slice(None), 0], in1, in_s1).wait()
        wait_flush()

    return k(x6, rd_inv, ld_inv)


def kernel(x):
    B, C, H, W = x.shape
    x6 = (
        jnp.transpose(x, (0, 2, 3, 1))
        .reshape(B, H, 4, 8, C // 128, 128)
        .transpose(0, 1, 2, 4, 3, 5)
    )
    yrd, yld = _diag_scan(x6, jnp.asarray(_RD_INV), jnp.asarray(_LD_INV), B, C)
    return yrd.reshape(B, C, HW), yld.reshape(B, C, HW)


# speedup vs baseline: 2.9562x; 1.0749x over previous
import functools

import jax
import jax.numpy as jnp
import numpy as np
from jax import lax
from jax.experimental import pallas as pl
from jax.experimental.pallas import tpu as pltpu
from jax.experimental.pallas import tpu_sc as plsc

H_DIM = 32
W_DIM = 32
HW = H_DIM * W_DIM

NC = 2
NS = 16
NW = NC * NS
L = 16

CPU = 32
HC = 4
S_CHUNK = HC * W_DIM
N_CHUNK = H_DIM // HC


def _diag_perm(mode):
    idx = []
    for d in range(H_DIM + W_DIM - 1):
        for i in range(H_DIM):
            j = d - i if mode == "rd" else i - (H_DIM - 1 - d)
            if 0 <= j < W_DIM:
                idx.append(i * W_DIM + j)
    return np.asarray(idx, dtype=np.int32)


def _inv(p):
    inv = np.empty_like(p)
    inv[p] = np.arange(p.size, dtype=p.dtype)
    return inv


_RD_INV = _inv(_diag_perm("rd"))
_LD_INV = _inv(_diag_perm("ld"))


@functools.partial(jax.jit, static_argnums=(3, 4))
def _diag_scan(x6, rd_inv, ld_inv, b_dim, c_dim):
    rows = b_dim * c_dim
    upb = c_dim // CPU
    steps = upb * N_CHUNK
    mesh = plsc.VectorSubcoreMesh(core_axis_name="c", subcore_axis_name="s")

    @functools.partial(
        pl.kernel,
        mesh=mesh,
        compiler_params=pltpu.CompilerParams(
            needs_layout_passes=False, use_tc_tiling_on_sc=True
        ),
        out_type=(
            jax.ShapeDtypeStruct((rows, HW), jnp.float32),
            jax.ShapeDtypeStruct((rows, HW), jnp.float32),
        ),
        scratch_types=[
            pltpu.VMEM((HW,), jnp.int32),
            pltpu.VMEM((HW,), jnp.int32),
            pltpu.VMEM((HC, 4, 8, 128), jnp.float32),
            pltpu.VMEM((HC, 4, 8, 128), jnp.float32),
            pltpu.VMEM((CPU, HW), jnp.float32),
            pltpu.VMEM((CPU, HW), jnp.float32),
            pltpu.SemaphoreType.DMA,
            pltpu.SemaphoreType.DMA,
            pltpu.SemaphoreType.DMA,
            pltpu.SemaphoreType.DMA,
        ],
    )
    def k(x_hbm, rdi_hbm, ldi_hbm, yrd_hbm, yld_hbm,
          rd_v, ld_v, in0, in1, or_v, ol_v,
          in_s0, in_s1, or_s, ol_s):
        b = lax.axis_index("s") * NC + lax.axis_index("c")
        pltpu.sync_copy(rdi_hbm, rd_v)
        pltpu.sync_copy(ldi_hbm, ld_v)

        def start_in(step, in_v, in_s):
            sc = jnp.minimum(step, steps - 1)
            cq = sc // N_CHUNK
            hc = sc % N_CHUNK
            pltpu.async_copy(
                x_hbm.at[b, pl.ds(hc * HC, HC), slice(None), cq // 4], in_v, in_s
            )

        iota = lax.iota(jnp.int32, L)

        def scatter_chunk(step, in_v):
            cq = step // N_CHUNK
            hc = step % N_CHUNK
            lane0 = (cq % 4) * CPU
            s0 = hc * S_CHUNK

            nrun = S_CHUNK // L
            krs = [rd_v[pl.ds(s0 + r * L, L)] for r in range(nrun)]
            kls = [ld_v[pl.ds(s0 + r * L, L)] for r in range(nrun)]
            iota128 = lax.shift_left(iota, 7)
            zero = jnp.zeros((L,), jnp.int32)

            @plsc.parallel_loop(0, CPU)
            def _(c0):
                cv = jnp.bitwise_and(c0 + iota, CPU - 1)
                base = cv + lane0 + iota128
                for r in range(nrun):
                    flat = base + jnp.int32(r * L * 128)
                    v = plsc.load_gather(in_v, [zero, zero, zero, flat])
                    plsc.store_scatter(or_v, [cv, krs[r]], v)
                    plsc.store_scatter(ol_v, [cv, kls[r]], v)

        def flush(cq):
            row0 = b * c_dim + cq * CPU
            pltpu.async_copy(or_v, yrd_hbm.at[pl.ds(row0, CPU)], or_s)
            pltpu.async_copy(ol_v, yld_hbm.at[pl.ds(row0, CPU)], ol_s)

        def wait_flush():
            pltpu.make_async_copy(or_v, yrd_hbm.at[pl.ds(0, CPU)], or_s).wait()
            pltpu.make_async_copy(ol_v, yld_hbm.at[pl.ds(0, CPU)], ol_s).wait()

        start_in(0, in0, in_s0)
        start_in(1, in1, in_s1)

        def half(step, in_v, in_s):
            pltpu.make_async_copy(
                x_hbm.at[b, pl.ds(0, HC), slice(None), 0], in_v, in_s
            ).wait()

            @pl.when(jnp.logical_and(step % N_CHUNK == 0, step > 0))
            def _():
                wait_flush()

            scatter_chunk(step, in_v)
            start_in(step + 2, in_v, in_s)

            @pl.when(step % N_CHUNK == N_CHUNK - 1)
            def _():
                flush(step // N_CHUNK)

        def body(i, carry):
            half(2 * i, in0, in_s0)
            half(2 * i + 1, in1, in_s1)
            return carry

        lax.fori_loop(0, steps // 2, body, 0)

        pltpu.make_async_copy(x_hbm.at[b, pl.ds(0, HC), slice(None), 0], in0, in_s0).wait()
        pltpu.make_async_copy(x_hbm.at[b, pl.ds(0, HC), slice(None), 0], in1, in_s1).wait()
        wait_flush()

    return k(x6, rd_inv, ld_inv)


def kernel(x):
    B, C, H, W = x.shape
    x6 = (
        jnp.transpose(x, (0, 2, 3, 1))
        .reshape(B, H, 4, 8, C // 128, 128)
        .transpose(0, 1, 2, 4, 3, 5)
    )
    yrd, yld = _diag_scan(x6, jnp.asarray(_RD_INV), jnp.asarray(_LD_INV), B, C)
    return yrd.reshape(B, C, HW), yld.reshape(B, C, HW)


# speedup vs baseline: 3.2888x; 1.1125x over previous
import functools

import jax
import jax.numpy as jnp
import numpy as np
from jax import lax
from jax.experimental import pallas as pl
from jax.experimental.pallas import tpu as pltpu
from jax.experimental.pallas import tpu_sc as plsc

H_DIM = 32
W_DIM = 32
HW = H_DIM * W_DIM

NC = 2
NS = 16
NW = NC * NS
L = 16

CPU = 32
HC = 4
S_CHUNK = HC * W_DIM
N_CHUNK = H_DIM // HC


def _diag_perm(mode):
    idx = []
    for d in range(H_DIM + W_DIM - 1):
        for i in range(H_DIM):
            j = d - i if mode == "rd" else i - (H_DIM - 1 - d)
            if 0 <= j < W_DIM:
                idx.append(i * W_DIM + j)
    return np.asarray(idx, dtype=np.int32)


def _inv(p):
    inv = np.empty_like(p)
    inv[p] = np.arange(p.size, dtype=p.dtype)
    return inv


_RD_INV = _inv(_diag_perm("rd"))
_LD_INV = _inv(_diag_perm("ld"))


@functools.partial(jax.jit, static_argnums=(3, 4))
def _diag_scan(x6, rd_inv, ld_inv, b_dim, c_dim):
    rows = b_dim * c_dim
    upb = c_dim // CPU
    steps = upb * N_CHUNK
    mesh = plsc.VectorSubcoreMesh(core_axis_name="c", subcore_axis_name="s")

    @functools.partial(
        pl.kernel,
        mesh=mesh,
        compiler_params=pltpu.CompilerParams(
            needs_layout_passes=False, use_tc_tiling_on_sc=True
        ),
        out_type=(
            jax.ShapeDtypeStruct((rows, HW), jnp.float32),
            jax.ShapeDtypeStruct((rows, HW), jnp.float32),
        ),
        scratch_types=[
            pltpu.VMEM((HW,), jnp.int32),
            pltpu.VMEM((HW,), jnp.int32),
            pltpu.VMEM((HC, 4, 8, 128), jnp.float32),
            pltpu.VMEM((HC, 4, 8, 128), jnp.float32),
            pltpu.VMEM((HC, 4, 8, 128), jnp.float32),
            pltpu.VMEM((CPU, HW), jnp.float32),
            pltpu.VMEM((CPU, HW), jnp.float32),
            pltpu.SemaphoreType.DMA,
            pltpu.SemaphoreType.DMA,
            pltpu.SemaphoreType.DMA,
            pltpu.SemaphoreType.DMA,
            pltpu.SemaphoreType.DMA,
        ],
    )
    def k(x_hbm, rdi_hbm, ldi_hbm, yrd_hbm, yld_hbm,
          rd_v, ld_v, in0, in1, in2, or_v, ol_v,
          in_s0, in_s1, in_s2, or_s, ol_s):
        b = lax.axis_index("s") * NC + lax.axis_index("c")
        pltpu.sync_copy(rdi_hbm, rd_v)
        pltpu.sync_copy(ldi_hbm, ld_v)

        def start_in(step, in_v, in_s):
            sc = jnp.minimum(step, steps - 1)
            cq = sc // N_CHUNK
            hc = sc % N_CHUNK
            pltpu.async_copy(
                x_hbm.at[b, pl.ds(hc * HC, HC), slice(None), cq // 4], in_v, in_s
            )

        iota = lax.iota(jnp.int32, L)

        def scatter_chunk(step, in_v):
            cq = step // N_CHUNK
            hc = step % N_CHUNK
            lane0 = (cq % 4) * CPU
            s0 = hc * S_CHUNK

            nrun = S_CHUNK // L
            krs = [rd_v[pl.ds(s0 + r * L, L)] for r in range(nrun)]
            kls = [ld_v[pl.ds(s0 + r * L, L)] for r in range(nrun)]
            iota128 = lax.shift_left(iota, 7)
            zero = jnp.zeros((L,), jnp.int32)

            @plsc.parallel_loop(0, CPU)
            def _(c0):
                cv = jnp.bitwise_and(c0 + iota, CPU - 1)
                base = cv + lane0 + iota128
                for r in range(nrun):
                    flat = base + jnp.int32(r * L * 128)
                    v = plsc.load_gather(in_v, [zero, zero, zero, flat])
                    plsc.store_scatter(or_v, [cv, krs[r]], v)
                    plsc.store_scatter(ol_v, [cv, kls[r]], v)

        def flush(cq):
            row0 = b * c_dim + cq * CPU
            pltpu.async_copy(or_v, yrd_hbm.at[pl.ds(row0, CPU)], or_s)
            pltpu.async_copy(ol_v, yld_hbm.at[pl.ds(row0, CPU)], ol_s)

        def wait_flush():
            pltpu.make_async_copy(or_v, yrd_hbm.at[pl.ds(0, CPU)], or_s).wait()
            pltpu.make_async_copy(ol_v, yld_hbm.at[pl.ds(0, CPU)], ol_s).wait()

        start_in(0, in0, in_s0)
        start_in(1, in1, in_s1)
        start_in(2, in2, in_s2)

        def half(step, in_v, in_s):
            pltpu.make_async_copy(
                x_hbm.at[b, pl.ds(0, HC), slice(None), 0], in_v, in_s
            ).wait()

            @pl.when(jnp.logical_and(step % N_CHUNK == 0, step > 0))
            def _():
                wait_flush()

            scatter_chunk(step, in_v)
            start_in(step + 3, in_v, in_s)

            @pl.when(step % N_CHUNK == N_CHUNK - 1)
            def _():
                flush(step // N_CHUNK)

        def body(i, carry):
            half(3 * i, in0, in_s0)
            half(3 * i + 1, in1, in_s1)
            half(3 * i + 2, in2, in_s2)
            return carry

        lax.fori_loop(0, steps // 3, body, 0)

        pltpu.make_async_copy(x_hbm.at[b, pl.ds(0, HC), slice(None), 0], in0, in_s0).wait()
        pltpu.make_async_copy(x_hbm.at[b, pl.ds(0, HC), slice(None), 0], in1, in_s1).wait()
        pltpu.make_async_copy(x_hbm.at[b, pl.ds(0, HC), slice(None), 0], in2, in_s2).wait()
        wait_flush()

    return k(x6, rd_inv, ld_inv)


def kernel(x):
    B, C, H, W = x.shape
    x6 = (
        jnp.transpose(x, (0, 2, 3, 1))
        .reshape(B, H, 4, 8, C // 128, 128)
        .transpose(0, 1, 2, 4, 3, 5)
    )
    yrd, yld = _diag_scan(x6, jnp.asarray(_RD_INV), jnp.asarray(_LD_INV), B, C)
    return yrd.reshape(B, C, HW), yld.reshape(B, C, HW)
